# baseline (speedup 1.0000x reference)
import functools

import jax
import jax.numpy as jnp
from jax import lax
from jax.experimental import pallas as pl
from jax.experimental.pallas import tpu as pltpu
from jax.experimental.pallas import tpu_sc as plsc

D = 64
D3 = 3 * D
B = 16384
REG = 0.01

NC = 2
NS = 16
NW = NC * NS
BPW = B // NW
CH = 32
NCH = BPW // CH
NT = 1000000 // 8


def _fire(tidx_v, table, stage, sem, c):
    pltpu.async_copy(table.at[tidx_v.at[c]], stage, sem)


def _extract(idx_v, table, out, stage, rows_v, sem, c, base):
    pltpu.make_async_copy(table.at[pl.ds(0, CH)], stage, sem).wait()
    for g in range(CH // 16):
        v = idx_v[pl.ds(c * CH + g * 16, 16)]
        subs = lax.bitwise_and(v, 7)
        for j in range(16):
            slot = g * 16 + j
            sub = subs[j]
            for k in range(D // 16):
                rows_v[slot, pl.ds(k * 16, 16)] = (
                    stage[slot, sub, pl.ds(k * 16, 16)])
    pltpu.sync_copy(rows_v, out.at[pl.ds(base + c * CH, CH)])


def _gather_body(idx_hbm, table, out, idx_v, tidx_v,
                 stage_a, stage_b, rows_a, rows_b, sem_a, sem_b):
    wid = lax.axis_index("s") * NC + lax.axis_index("c")
    base = wid * BPW
    pltpu.sync_copy(idx_hbm.at[pl.ds(base, BPW)], idx_v)
    for t in range(BPW // 16):
        v = idx_v[pl.ds(t * 16, 16)]
        tidx_v[t // (CH // 16), pl.ds((t % (CH // 16)) * 16, 16)] = (
            lax.shift_right_logical(v, 3))
    _fire(tidx_v, table, stage_a, sem_a, 0)

    def body(d, _):
        c0 = 2 * d
        _fire(tidx_v, table, stage_b, sem_b, c0 + 1)
        _extract(idx_v, table, out, stage_a, rows_a, sem_a, c0, base)

        @pl.when(d + 1 < NCH // 2)
        def _():
            _fire(tidx_v, table, stage_a, sem_a, c0 + 2)

        _extract(idx_v, table, out, stage_b, rows_b, sem_b, c0 + 1, base)
        return 0

    lax.fori_loop(0, NCH // 2, body, 0)


@functools.cache
def _sc_gather():
    return pl.kernel(
        _gather_body,
        out_type=jax.ShapeDtypeStruct((B, D), jnp.float32),
        mesh=plsc.VectorSubcoreMesh(core_axis_name="c", subcore_axis_name="s"),
        scratch_types=[
            pltpu.VMEM((BPW,), jnp.int32),
            pltpu.VMEM((NCH, CH), jnp.int32),
            pltpu.VMEM((CH, 8, D), jnp.float32),
            pltpu.VMEM((CH, 8, D), jnp.float32),
            pltpu.VMEM((CH, D), jnp.float32),
            pltpu.VMEM((CH, D), jnp.float32),
            pltpu.SemaphoreType.DMA,
            pltpu.SemaphoreType.DMA,
        ],
        compiler_params=pltpu.CompilerParams(use_tc_tiling_on_sc=False),
    )


BB = 4096
NB = B // BB


def _mlp_body(xp_ref, xq_ref, xr_ref, w1t_ref, b1_ref, w2t_ref, b2_ref,
              inf_ref, regs_ref, acc_ref):
    i = pl.program_id(0)
    w1t = w1t_ref[...]
    n1 = jnp.sqrt(jnp.sum(w1t * w1t, axis=0, keepdims=True))
    w1tc = w1t / jnp.maximum(n1, 1.0)
    w2t = w2t_ref[...]
    n2 = jnp.sqrt(jnp.sum(w2t * w2t))
    w2tc = w2t / jnp.maximum(n2, 1.0)

    xp = xp_ref[...]
    xq = xq_ref[...]
    xr = xr_ref[...]

    dn = (((1,), (0,)), ((), ()))
    x = jnp.concatenate([xp, xq, xr], axis=1)
    acc = lax.dot_general(x, w1tc, dn,
                          preferred_element_type=jnp.float32)
    h = jax.nn.sigmoid(acc + b1_ref[...])
    inf = lax.dot_general(h, w2tc, dn,
                          preferred_element_type=jnp.float32)
    inf_ref[...] = inf + b2_ref[...]

    sp = jnp.sum(xp * xp)
    sq = jnp.sum(xq * xq)
    sr = jnp.sum(xr * xr)

    @pl.when(i == 0)
    def _():
        acc_ref[0] = sp
        acc_ref[1] = sq
        acc_ref[2] = sr

    @pl.when(i > 0)
    def _():
        acc_ref[0] += sp
        acc_ref[1] += sq
        acc_ref[2] += sr

    @pl.when(i == NB - 1)
    def _():
        regs = REG * (jnp.sqrt(acc_ref[0]) + jnp.sqrt(acc_ref[1]) +
                      jnp.sqrt(acc_ref[2]))
        regs_ref[...] = jnp.broadcast_to(regs, (1, 1))


def _mlp(xp, xq, xr, W1t, b1_row, W2t, b2_2d):
    x_spec = pl.BlockSpec((BB, D), lambda i: (i, 0))
    return pl.pallas_call(
        _mlp_body,
        grid=(NB,),
        in_specs=[x_spec, x_spec, x_spec,
                  pl.BlockSpec((D3, D3), lambda i: (0, 0)),
                  pl.BlockSpec((1, D3), lambda i: (0, 0)),
                  pl.BlockSpec((D3, 1), lambda i: (0, 0)),
                  pl.BlockSpec((1, 1), lambda i: (0, 0))],
        out_specs=(pl.BlockSpec((BB, 1), lambda i: (i, 0)),
                   pl.BlockSpec((1, 1), lambda i: (0, 0))),
        out_shape=(jax.ShapeDtypeStruct((B, 1), jnp.float32),
                   jax.ShapeDtypeStruct((1, 1), jnp.float32)),
        scratch_shapes=[pltpu.SMEM((3,), jnp.float32)],
    )(xp, xq, xr, W1t, b1_row, W2t, b2_2d)


def kernel(ps, qs, rs, table_p, table_q, table_r, W1, b1, W2, b2):
    g = _sc_gather()
    xp = g(ps.astype(jnp.int32), table_p.reshape(NT, 8, D))
    xq = g(qs.astype(jnp.int32), table_q.reshape(NT, 8, D))
    xr = g(rs.astype(jnp.int32), table_r.reshape(NT, 8, D))
    inf, regs = _mlp(xp, xq, xr, W1.T, b1.reshape(1, D3), W2.T,
                     b2.reshape(1, 1))
    return inf, regs[0, 0]


# speedup vs baseline: 2.3319x; 2.3319x over previous
import functools

import jax
import jax.numpy as jnp
from jax import lax
from jax.experimental import pallas as pl
from jax.experimental.pallas import tpu as pltpu
from jax.experimental.pallas import tpu_sc as plsc

D = 64
D3 = 3 * D
B = 16384
REG = 0.01

NC = 2
NS = 16
NW = NC * NS
BPW = B // NW
CH = 32
NCH = BPW // CH
NT = 1000000 // 8


def _gather_body(idx_hbm, table, out, idx_v, rows_v, sem):
    wid = lax.axis_index("s") * NC + lax.axis_index("c")
    base = wid * BPW
    pltpu.sync_copy(idx_hbm.at[pl.ds(base, BPW)], idx_v)

    def fire(g, _):
        v = idx_v[pl.ds(g * 16, 16)]
        ts = lax.shift_right_logical(v, 3)
        ss = lax.bitwise_and(v, 7)
        for j in range(16):
            pltpu.make_async_copy(
                table.at[pl.ds(ts[j], 1), pl.ds(ss[j], 1)],
                rows_v.at[pl.ds(g * 16 + j, 1)], sem).start()
        return 0

    lax.fori_loop(0, BPW // 16, fire, 0)
    pltpu.make_async_copy(table.at[pl.ds(0, BPW), pl.ds(0, 1)], rows_v,
                          sem).wait()
    pltpu.sync_copy(rows_v, out.at[pl.ds(base, BPW)])


@functools.cache
def _sc_gather():
    return pl.kernel(
        _gather_body,
        out_type=jax.ShapeDtypeStruct((B, 1, D), jnp.float32),
        mesh=plsc.VectorSubcoreMesh(core_axis_name="c", subcore_axis_name="s"),
        scratch_types=[
            pltpu.VMEM((BPW,), jnp.int32),
            pltpu.VMEM((BPW, 1, D), jnp.float32),
            pltpu.SemaphoreType.DMA,
        ],
    )


BB = 4096
NB = B // BB


def _mlp_body(xp_ref, xq_ref, xr_ref, w1t_ref, b1_ref, w2t_ref, b2_ref,
              inf_ref, regs_ref, acc_ref):
    i = pl.program_id(0)
    w1t = w1t_ref[...]
    n1 = jnp.sqrt(jnp.sum(w1t * w1t, axis=0, keepdims=True))
    w1tc = w1t / jnp.maximum(n1, 1.0)
    w2t = w2t_ref[...]
    n2 = jnp.sqrt(jnp.sum(w2t * w2t))
    w2tc = w2t / jnp.maximum(n2, 1.0)

    xp = xp_ref[...]
    xq = xq_ref[...]
    xr = xr_ref[...]

    dn = (((1,), (0,)), ((), ()))
    x = jnp.concatenate([xp, xq, xr], axis=1)
    acc = lax.dot_general(x, w1tc, dn,
                          preferred_element_type=jnp.float32)
    h = jax.nn.sigmoid(acc + b1_ref[...])
    inf = lax.dot_general(h, w2tc, dn,
                          preferred_element_type=jnp.float32)
    inf_ref[...] = inf + b2_ref[...]

    sp = jnp.sum(xp * xp)
    sq = jnp.sum(xq * xq)
    sr = jnp.sum(xr * xr)

    @pl.when(i == 0)
    def _():
        acc_ref[0] = sp
        acc_ref[1] = sq
        acc_ref[2] = sr

    @pl.when(i > 0)
    def _():
        acc_ref[0] += sp
        acc_ref[1] += sq
        acc_ref[2] += sr

    @pl.when(i == NB - 1)
    def _():
        regs = REG * (jnp.sqrt(acc_ref[0]) + jnp.sqrt(acc_ref[1]) +
                      jnp.sqrt(acc_ref[2]))
        regs_ref[...] = jnp.broadcast_to(regs, (1, 1))


def _mlp(xp, xq, xr, W1t, b1_row, W2t, b2_2d):
    x_spec = pl.BlockSpec((BB, D), lambda i: (i, 0))
    return pl.pallas_call(
        _mlp_body,
        grid=(NB,),
        in_specs=[x_spec, x_spec, x_spec,
                  pl.BlockSpec((D3, D3), lambda i: (0, 0)),
                  pl.BlockSpec((1, D3), lambda i: (0, 0)),
                  pl.BlockSpec((D3, 1), lambda i: (0, 0)),
                  pl.BlockSpec((1, 1), lambda i: (0, 0))],
        out_specs=(pl.BlockSpec((BB, 1), lambda i: (i, 0)),
                   pl.BlockSpec((1, 1), lambda i: (0, 0))),
        out_shape=(jax.ShapeDtypeStruct((B, 1), jnp.float32),
                   jax.ShapeDtypeStruct((1, 1), jnp.float32)),
        scratch_shapes=[pltpu.SMEM((3,), jnp.float32)],
    )(xp, xq, xr, W1t, b1_row, W2t, b2_2d)


def kernel(ps, qs, rs, table_p, table_q, table_r, W1, b1, W2, b2):
    g = _sc_gather()
    xp = g(ps.astype(jnp.int32), table_p.reshape(NT, 8, D)).reshape(B, D)
    xq = g(qs.astype(jnp.int32), table_q.reshape(NT, 8, D)).reshape(B, D)
    xr = g(rs.astype(jnp.int32), table_r.reshape(NT, 8, D)).reshape(B, D)
    inf, regs = _mlp(xp, xq, xr, W1.T, b1.reshape(1, D3), W2.T,
                     b2.reshape(1, 1))
    return inf, regs[0, 0]


# speedup vs baseline: 2.3550x; 1.0099x over previous
import functools

import jax
import jax.numpy as jnp
from jax import lax
from jax.experimental import pallas as pl
from jax.experimental.pallas import tpu as pltpu
from jax.experimental.pallas import tpu_sc as plsc

D = 64
D3 = 3 * D
B = 16384
REG = 0.01

NC = 2
NS = 16
NW = NC * NS
BPW = B // NW
CH = 32
NCH = BPW // CH
NT = 1000000 // 8


def _gather_body(idx_p, idx_q, idx_r, tab_p, tab_q, tab_r,
                 out_p, out_q, out_r, idx_v, rows_v, sem):
    wid = lax.axis_index("s") * NC + lax.axis_index("c")
    base = wid * BPW
    for idx_hbm, table, out in ((idx_p, tab_p, out_p),
                                (idx_q, tab_q, out_q),
                                (idx_r, tab_r, out_r)):
        pltpu.sync_copy(idx_hbm.at[pl.ds(base, BPW)], idx_v)

        def fire(g, _, table=table):
            v = idx_v[pl.ds(g * 16, 16)]
            ts = lax.shift_right_logical(v, 3)
            ss = lax.bitwise_and(v, 7)
            for j in range(16):
                pltpu.make_async_copy(
                    table.at[pl.ds(ts[j], 1), pl.ds(ss[j], 1)],
                    rows_v.at[pl.ds(g * 16 + j, 1)], sem).start()
            return 0

        lax.fori_loop(0, BPW // 16, fire, 0)
        pltpu.make_async_copy(table.at[pl.ds(0, BPW), pl.ds(0, 1)], rows_v,
                              sem).wait()
        pltpu.sync_copy(rows_v, out.at[pl.ds(base, BPW)])


@functools.cache
def _sc_gather():
    return pl.kernel(
        _gather_body,
        out_type=[jax.ShapeDtypeStruct((B, 1, D), jnp.float32)] * 3,
        mesh=plsc.VectorSubcoreMesh(core_axis_name="c", subcore_axis_name="s"),
        scratch_types=[
            pltpu.VMEM((BPW,), jnp.int32),
            pltpu.VMEM((BPW, 1, D), jnp.float32),
            pltpu.SemaphoreType.DMA,
        ],
    )


BB = 4096
NB = B // BB


def _mlp_body(xp_ref, xq_ref, xr_ref, w1t_ref, b1_ref, w2t_ref, b2_ref,
              inf_ref, regs_ref, acc_ref):
    i = pl.program_id(0)
    w1t = w1t_ref[...]
    n1 = jnp.sqrt(jnp.sum(w1t * w1t, axis=0, keepdims=True))
    w1tc = w1t / jnp.maximum(n1, 1.0)
    w2t = w2t_ref[...]
    n2 = jnp.sqrt(jnp.sum(w2t * w2t))
    w2tc = w2t / jnp.maximum(n2, 1.0)

    xp = xp_ref[...]
    xq = xq_ref[...]
    xr = xr_ref[...]

    dn = (((1,), (0,)), ((), ()))
    x = jnp.concatenate([xp, xq, xr], axis=1)
    acc = lax.dot_general(x, w1tc, dn,
                          preferred_element_type=jnp.float32)
    h = jax.nn.sigmoid(acc + b1_ref[...])
    inf = lax.dot_general(h, w2tc, dn,
                          preferred_element_type=jnp.float32)
    inf_ref[...] = inf + b2_ref[...]

    sp = jnp.sum(xp * xp)
    sq = jnp.sum(xq * xq)
    sr = jnp.sum(xr * xr)

    @pl.when(i == 0)
    def _():
        acc_ref[0] = sp
        acc_ref[1] = sq
        acc_ref[2] = sr

    @pl.when(i > 0)
    def _():
        acc_ref[0] += sp
        acc_ref[1] += sq
        acc_ref[2] += sr

    @pl.when(i == NB - 1)
    def _():
        regs = REG * (jnp.sqrt(acc_ref[0]) + jnp.sqrt(acc_ref[1]) +
                      jnp.sqrt(acc_ref[2]))
        regs_ref[...] = jnp.broadcast_to(regs, (1, 1))


def _mlp(xp, xq, xr, W1t, b1_row, W2t, b2_2d):
    x_spec = pl.BlockSpec((BB, D), lambda i: (i, 0))
    return pl.pallas_call(
        _mlp_body,
        grid=(NB,),
        in_specs=[x_spec, x_spec, x_spec,
                  pl.BlockSpec((D3, D3), lambda i: (0, 0)),
                  pl.BlockSpec((1, D3), lambda i: (0, 0)),
                  pl.BlockSpec((D3, 1), lambda i: (0, 0)),
                  pl.BlockSpec((1, 1), lambda i: (0, 0))],
        out_specs=(pl.BlockSpec((BB, 1), lambda i: (i, 0)),
                   pl.BlockSpec((1, 1), lambda i: (0, 0))),
        out_shape=(jax.ShapeDtypeStruct((B, 1), jnp.float32),
                   jax.ShapeDtypeStruct((1, 1), jnp.float32)),
        scratch_shapes=[pltpu.SMEM((3,), jnp.float32)],
    )(xp, xq, xr, W1t, b1_row, W2t, b2_2d)


def kernel(ps, qs, rs, table_p, table_q, table_r, W1, b1, W2, b2):
    xp3, xq3, xr3 = _sc_gather()(
        ps.astype(jnp.int32), qs.astype(jnp.int32), rs.astype(jnp.int32),
        table_p.reshape(NT, 8, D), table_q.reshape(NT, 8, D),
        table_r.reshape(NT, 8, D))
    xp = xp3.reshape(B, D)
    xq = xq3.reshape(B, D)
    xr = xr3.reshape(B, D)
    inf, regs = _mlp(xp, xq, xr, W1.T, b1.reshape(1, D3), W2.T,
                     b2.reshape(1, 1))
    return inf, regs[0, 0]


# speedup vs baseline: 2.3582x; 1.0014x over previous
import functools

import jax
import jax.numpy as jnp
from jax import lax
from jax.experimental import pallas as pl
from jax.experimental.pallas import tpu as pltpu
from jax.experimental.pallas import tpu_sc as plsc

D = 64
D3 = 3 * D
B = 16384
REG = 0.01

NC = 2
NS = 16
NW = NC * NS
BPW = B // NW
NT = 1000000 // 8


def _gather_body(idx_p, idx_q, idx_r, tab_p, tab_q, tab_r,
                 out_p, out_q, out_r, idx_v, rows_v, sem):
    wid = lax.axis_index("s") * NC + lax.axis_index("c")
    base = wid * BPW
    for idx_hbm, table, out in ((idx_p, tab_p, out_p),
                                (idx_q, tab_q, out_q),
                                (idx_r, tab_r, out_r)):
        pltpu.sync_copy(idx_hbm.at[pl.ds(base, BPW)], idx_v)

        def fire(g, _, table=table):
            v = idx_v[pl.ds(g * 16, 16)]
            ts = lax.shift_right_logical(v, 3)
            ss = lax.bitwise_and(v, 7)
            for j in range(16):
                pltpu.make_async_copy(
                    table.at[pl.ds(ts[j], 1), pl.ds(ss[j], 1)],
                    rows_v.at[pl.ds(g * 16 + j, 1)], sem).start()
            return 0

        lax.fori_loop(0, BPW // 16, fire, 0)
        pltpu.make_async_copy(table.at[pl.ds(0, BPW), pl.ds(0, 1)], rows_v,
                              sem).wait()
        pltpu.sync_copy(rows_v, out.at[pl.ds(base, BPW)])


@functools.cache
def _sc_gather():
    return pl.kernel(
        _gather_body,
        out_type=[jax.ShapeDtypeStruct((B, 1, D), jnp.float32)] * 3,
        mesh=plsc.VectorSubcoreMesh(core_axis_name="c", subcore_axis_name="s"),
        scratch_types=[
            pltpu.VMEM((BPW,), jnp.int32),
            pltpu.VMEM((BPW, 1, D), jnp.float32),
            pltpu.SemaphoreType.DMA,
        ],
    )


BB = 4096
NB = B // BB


def _mlp_body(xp_ref, xq_ref, xr_ref, w1t_ref, b1_ref, w2t_ref, b2_ref,
              inf_ref, regs_ref, acc_ref):
    i = pl.program_id(0)
    w1t = w1t_ref[...]
    n1 = jnp.sqrt(jnp.sum(w1t * w1t, axis=0, keepdims=True))
    w1tc = w1t / jnp.maximum(n1, 1.0)
    w2t = w2t_ref[...]
    n2 = jnp.sqrt(jnp.sum(w2t * w2t))
    w2tc = w2t / jnp.maximum(n2, 1.0)

    xp = xp_ref[...]
    xq = xq_ref[...]
    xr = xr_ref[...]

    dn = (((1,), (0,)), ((), ()))
    x = jnp.concatenate([xp, xq, xr], axis=1)
    acc = lax.dot_general(x, w1tc, dn,
                          preferred_element_type=jnp.float32)
    h = jax.nn.sigmoid(acc + b1_ref[...])
    inf = lax.dot_general(h, w2tc, dn,
                          preferred_element_type=jnp.float32)
    inf_ref[...] = inf + b2_ref[...]

    sp = jnp.sum(xp * xp)
    sq = jnp.sum(xq * xq)
    sr = jnp.sum(xr * xr)

    @pl.when(i == 0)
    def _():
        acc_ref[0] = sp
        acc_ref[1] = sq
        acc_ref[2] = sr

    @pl.when(i > 0)
    def _():
        acc_ref[0] += sp
        acc_ref[1] += sq
        acc_ref[2] += sr

    @pl.when(i == NB - 1)
    def _():
        regs = REG * (jnp.sqrt(acc_ref[0]) + jnp.sqrt(acc_ref[1]) +
                      jnp.sqrt(acc_ref[2]))
        regs_ref[...] = jnp.broadcast_to(regs, (1, 1))


def _mlp(xp, xq, xr, W1t, b1_row, W2t, b2_2d):
    x_spec = pl.BlockSpec((BB, D), lambda i: (i, 0))
    return pl.pallas_call(
        _mlp_body,
        grid=(NB,),
        in_specs=[x_spec, x_spec, x_spec,
                  pl.BlockSpec((D3, D3), lambda i: (0, 0)),
                  pl.BlockSpec((1, D3), lambda i: (0, 0)),
                  pl.BlockSpec((D3, 1), lambda i: (0, 0)),
                  pl.BlockSpec((1, 1), lambda i: (0, 0))],
        out_specs=(pl.BlockSpec((BB, 1), lambda i: (i, 0)),
                   pl.BlockSpec((1, 1), lambda i: (0, 0))),
        out_shape=(jax.ShapeDtypeStruct((B, 1), jnp.float32),
                   jax.ShapeDtypeStruct((1, 1), jnp.float32)),
        scratch_shapes=[pltpu.SMEM((3,), jnp.float32)],
    )(xp, xq, xr, W1t, b1_row, W2t, b2_2d)


def kernel(ps, qs, rs, table_p, table_q, table_r, W1, b1, W2, b2):
    xp3, xq3, xr3 = _sc_gather()(
        ps.astype(jnp.int32), qs.astype(jnp.int32), rs.astype(jnp.int32),
        table_p.reshape(NT, 8, D), table_q.reshape(NT, 8, D),
        table_r.reshape(NT, 8, D))
    xp = xp3.reshape(B, D)
    xq = xq3.reshape(B, D)
    xr = xr3.reshape(B, D)
    inf, regs = _mlp(xp, xq, xr, W1.T, b1.reshape(1, D3), W2.T,
                     b2.reshape(1, 1))
    return inf, regs[0, 0]
